# Initial kernel scaffold; baseline (speedup 1.0000x reference)
import jax
import jax.numpy as jnp
from jax.experimental import pallas as pl


def kernel(x, edge_index, batch, W0, b0, W1, b1, W2, b2, fc1_W, fc1_b, fc2_W, fc2_b):
    raise NotImplementedError("write your pallas kernel here")


# speedup vs baseline: 7.5130x; 7.5130x over previous
import functools

import jax
import jax.numpy as jnp
from jax import lax
from jax.experimental import pallas as pl
from jax.experimental.pallas import tpu as pltpu
from jax.experimental.pallas import tpu_sc as plsc

N = 10000
NP = 10240
D = 128
E = 320000
EP = 327680
G = 16
NC = 2
NS = 16
CH = 128
NCH = EP // (NC * NS * CH)
RS = NP // NS
BR = 512
F32 = jnp.float32

@functools.lru_cache(maxsize=1)
def _sc_kernels():
    mesh = plsc.VectorSubcoreMesh(core_axis_name="c", subcore_axis_name="s")

    @functools.partial(
        pl.kernel,
        mesh=mesh,
        out_type=jax.ShapeDtypeStruct((NC, NP, D), F32),
        scratch_types=[
            pltpu.VMEM((NCH, CH), jnp.int32),
            pltpu.VMEM((CH, D), F32),
            pltpu.VMEM_SHARED((NP, D), F32),
        ],
    )
    def _deg_sc(dst_hbm, ones_hbm, z_hbm, out_hbm, idx_v, ones_v, acc_sh):
        c = lax.axis_index("c")
        s = lax.axis_index("s")
        wid = c * NS + s
        pltpu.sync_copy(dst_hbm.at[wid], idx_v)
        pltpu.sync_copy(ones_hbm, ones_v)
        pltpu.sync_copy(z_hbm.at[pl.ds(s * RS, RS)],
                        acc_sh.at[pl.ds(s * RS, RS)])
        plsc.subcore_barrier()

        def body(i, carry):
            pltpu.sync_copy(ones_v, acc_sh.at[idx_v.at[i]], add=True)
            return carry

        lax.fori_loop(0, NCH, body, 0)
        plsc.subcore_barrier()
        pltpu.sync_copy(acc_sh.at[pl.ds(s * RS, RS)],
                        out_hbm.at[c].at[pl.ds(s * RS, RS)])

    @functools.partial(
        pl.kernel,
        mesh=mesh,
        out_type=jax.ShapeDtypeStruct((NC, NP, D), F32),
        scratch_types=[
            pltpu.VMEM((NCH, CH), jnp.int32),
            pltpu.VMEM((NCH, CH), jnp.int32),
            pltpu.VMEM((CH, D), F32),
            pltpu.VMEM_SHARED((NP, D), F32),
            pltpu.SemaphoreType.DMA,
        ],
    )
    def _prop_sc(u_hbm, src_hbm, dst_hbm, z_hbm, out_hbm,
                 src_v, dst_v, rows_v, acc_sh, sem):
        c = lax.axis_index("c")
        s = lax.axis_index("s")
        wid = c * NS + s
        pltpu.sync_copy(src_hbm.at[wid], src_v)
        pltpu.sync_copy(dst_hbm.at[wid], dst_v)
        pltpu.sync_copy(z_hbm.at[pl.ds(s * RS, RS)],
                        acc_sh.at[pl.ds(s * RS, RS)])
        plsc.subcore_barrier()

        def body(i, carry):
            pltpu.async_copy(u_hbm.at[src_v.at[i]], rows_v, sem).wait()
            pltpu.sync_copy(rows_v, acc_sh.at[dst_v.at[i]], add=True)
            return carry

        lax.fori_loop(0, NCH, body, 0)
        plsc.subcore_barrier()
        pltpu.sync_copy(acc_sh.at[pl.ds(s * RS, RS)],
                        out_hbm.at[c].at[pl.ds(s * RS, RS)])

    return _deg_sc, _prop_sc


def _prep_body(p16_ref, x_ref, w_ref, u_ref, dis_ref):
    p = p16_ref[...]
    cnt = p[0, :, :1] + p[1, :, :1]
    dis = lax.rsqrt(cnt + 1.0)
    u_ref[...] = jnp.dot(x_ref[...] * dis, w_ref[...],
                         preferred_element_type=F32)
    dis_ref[...] = dis


def _prep_tc(p16, xp, w0):
    return pl.pallas_call(
        _prep_body,
        grid=(NP // BR,),
        in_specs=[
            pl.BlockSpec((NC, BR, D), lambda i: (0, i, 0)),
            pl.BlockSpec((BR, D), lambda i: (i, 0)),
            pl.BlockSpec((D, D), lambda i: (0, 0)),
        ],
        out_specs=[
            pl.BlockSpec((BR, D), lambda i: (i, 0)),
            pl.BlockSpec((BR, 1), lambda i: (i, 0)),
        ],
        out_shape=[
            jax.ShapeDtypeStruct((NP, D), F32),
            jax.ShapeDtypeStruct((NP, 1), F32),
        ],
    )(p16, xp, w0)


def _layer_body(s_ref, u_ref, dis_ref, b_ref, w_ref, o_ref):
    sarr = s_ref[...]
    dis = dis_ref[...]
    u = u_ref[...]
    h = jnp.maximum(dis * (sarr[0] + sarr[1]) + dis * u + b_ref[...], 0.0)
    o_ref[...] = jnp.dot(dis * h, w_ref[...], preferred_element_type=F32)


def _layer_tc(s, u, dis, b, w_next):
    return pl.pallas_call(
        _layer_body,
        grid=(NP // BR,),
        in_specs=[
            pl.BlockSpec((NC, BR, D), lambda i: (0, i, 0)),
            pl.BlockSpec((BR, D), lambda i: (i, 0)),
            pl.BlockSpec((BR, 1), lambda i: (i, 0)),
            pl.BlockSpec((1, D), lambda i: (0, 0)),
            pl.BlockSpec((D, D), lambda i: (0, 0)),
        ],
        out_specs=pl.BlockSpec((BR, D), lambda i: (i, 0)),
        out_shape=jax.ShapeDtypeStruct((NP, D), F32),
    )(s, u, dis, b, w_next)


def _final_body(s_ref, u_ref, dis_ref, b_ref, bt_ref,
                f1w_ref, f1b_ref, f2w_ref, f2b_ref,
                o_ref, pooled, cnts):
    i = pl.program_id(0)

    @pl.when(i == 0)
    def _():
        pooled[...] = jnp.zeros_like(pooled)
        cnts[...] = jnp.zeros_like(cnts)

    sarr = s_ref[...]
    dis = dis_ref[...]
    u = u_ref[...]
    h = jnp.maximum(dis * (sarr[0] + sarr[1]) + dis * u + b_ref[...], 0.0)
    bt = bt_ref[...]
    m = (bt == lax.broadcasted_iota(jnp.int32, (BR, G), 1)).astype(F32)
    pooled[...] += lax.dot_general(m, h, (((0,), (0,)), ((), ())),
                                   preferred_element_type=F32)
    cnts[...] += lax.dot_general(m, jnp.ones_like(h), (((0,), (0,)), ((), ())),
                                 preferred_element_type=F32)

    @pl.when(i == NP // BR - 1)
    def _():
        pm = pooled[...] / jnp.maximum(cnts[...], 1.0)
        z = jnp.maximum(jnp.dot(pm, f1w_ref[...],
                                preferred_element_type=F32) + f1b_ref[...],
                        0.0)
        o_ref[...] = jnp.sum(z * f2w_ref[...], axis=1, keepdims=True) \
            + f2b_ref[...]


def _final_tc(s, u, dis, b, batchp, f1w, f1b, f2w, f2b):
    return pl.pallas_call(
        _final_body,
        grid=(NP // BR,),
        in_specs=[
            pl.BlockSpec((NC, BR, D), lambda i: (0, i, 0)),
            pl.BlockSpec((BR, D), lambda i: (i, 0)),
            pl.BlockSpec((BR, 1), lambda i: (i, 0)),
            pl.BlockSpec((1, D), lambda i: (0, 0)),
            pl.BlockSpec((BR, 1), lambda i: (i, 0)),
            pl.BlockSpec((D, D), lambda i: (0, 0)),
            pl.BlockSpec((1, D), lambda i: (0, 0)),
            pl.BlockSpec((1, D), lambda i: (0, 0)),
            pl.BlockSpec((1, 1), lambda i: (0, 0)),
        ],
        out_specs=pl.BlockSpec((G, 1), lambda i: (0, 0)),
        out_shape=jax.ShapeDtypeStruct((G, 1), F32),
        scratch_shapes=[
            pltpu.VMEM((G, D), F32),
            pltpu.VMEM((G, D), F32),
        ],
    )(s, u, dis, b, batchp, f1w, f1b, f2w, f2b)


def kernel(x, edge_index, batch, W0, b0, W1, b1, W2, b2,
           fc1_W, fc1_b, fc2_W, fc2_b):
    src = edge_index[0]
    dst = edge_index[1]
    padi = jnp.full((EP - E,), N, jnp.int32)
    srcp = jnp.concatenate([src, padi]).reshape(NC * NS, NCH, CH)
    dstp = jnp.concatenate([dst, padi]).reshape(NC * NS, NCH, CH)

    xp = jnp.pad(x, ((0, NP - N), (0, 0)))
    zeros128 = jnp.zeros((NP, D), F32)
    ones128 = jnp.ones((CH, D), F32)
    batchp = jnp.pad(batch, (0, NP - N), constant_values=G).reshape(NP, 1)

    b0r = b0.reshape(1, D)
    b1r = b1.reshape(1, D)
    b2r = b2.reshape(1, D)
    f1b = fc1_b.reshape(1, D)
    f2w = fc2_W.reshape(1, D)
    f2b = fc2_b.reshape(1, 1)

    _deg_sc, _prop_sc = _sc_kernels()
    p16 = _deg_sc(dstp, ones128, zeros128)
    u0, dis = _prep_tc(p16, xp, W0)
    s0 = _prop_sc(u0, srcp, dstp, zeros128)
    u1 = _layer_tc(s0, u0, dis, b0r, W1)
    s1 = _prop_sc(u1, srcp, dstp, zeros128)
    u2 = _layer_tc(s1, u1, dis, b1r, W2)
    s2 = _prop_sc(u2, srcp, dstp, zeros128)
    return _final_tc(s2, u2, dis, b2r, batchp, fc1_W, f1b, f2w, f2b)


# speedup vs baseline: 7.7263x; 1.0284x over previous
import functools

import jax
import jax.numpy as jnp
from jax import lax
from jax.experimental import pallas as pl
from jax.experimental.pallas import tpu as pltpu
from jax.experimental.pallas import tpu_sc as plsc

N = 10000
NP = 10240
D = 128
E = 320000
EP = 327680
G = 16
NC = 2
NS = 16
CH = 128
NCH = EP // (NC * NS * CH)
CHP = 64
NCHP = EP // (NC * NS * CHP)
NB = 4
NBLK = NCHP // NB
RS = NP // NS
BR = 512
F32 = jnp.float32

@functools.lru_cache(maxsize=1)
def _sc_kernels():
    mesh = plsc.VectorSubcoreMesh(core_axis_name="c", subcore_axis_name="s")

    @functools.partial(
        pl.kernel,
        mesh=mesh,
        out_type=jax.ShapeDtypeStruct((NC, NP, D), F32),
        scratch_types=[
            pltpu.VMEM((NCH, CH), jnp.int32),
            pltpu.VMEM((CH, D), F32),
            pltpu.VMEM_SHARED((NP, D), F32),
        ],
    )
    def _deg_sc(dst_hbm, ones_hbm, z_hbm, out_hbm, idx_v, ones_v, acc_sh):
        c = lax.axis_index("c")
        s = lax.axis_index("s")
        wid = c * NS + s
        pltpu.sync_copy(dst_hbm.at[wid], idx_v)
        pltpu.sync_copy(ones_hbm, ones_v)
        pltpu.sync_copy(z_hbm.at[pl.ds(s * RS, RS)],
                        acc_sh.at[pl.ds(s * RS, RS)])
        plsc.subcore_barrier()

        def body(i, carry):
            pltpu.sync_copy(ones_v, acc_sh.at[idx_v.at[i]], add=True)
            return carry

        lax.fori_loop(0, NCH, body, 0)
        plsc.subcore_barrier()
        pltpu.sync_copy(acc_sh.at[pl.ds(s * RS, RS)],
                        out_hbm.at[c].at[pl.ds(s * RS, RS)])

    @functools.partial(
        pl.kernel,
        mesh=mesh,
        out_type=jax.ShapeDtypeStruct((NC, NP, D), F32),
        scratch_types=[
            pltpu.VMEM((2, NB, CHP), jnp.int32),
            pltpu.VMEM((2, NB, CHP), jnp.int32),
            pltpu.VMEM((NB, CHP, D), F32),
            pltpu.VMEM_SHARED((NP, D), F32),
        ] + [pltpu.SemaphoreType.DMA] * (NB + 2),
    )
    def _prop_sc(u_hbm, src_hbm, dst_hbm, z_hbm, out_hbm,
                 sidx, didx, rows_v, acc_sh, *sems):
        isems = sems[NB:]
        c = lax.axis_index("c")
        s = lax.axis_index("s")
        wid = c * NS + s
        eh_s = src_hbm.at[wid]
        eh_d = dst_hbm.at[wid]

        def load_idx(b, buf):
            pltpu.async_copy(eh_s.at[pl.ds(b * NB, NB)], sidx.at[buf],
                             isems[buf])
            pltpu.async_copy(eh_d.at[pl.ds(b * NB, NB)], didx.at[buf],
                             isems[buf])

        def wait_idx(buf):
            pltpu.make_async_copy(eh_s.at[pl.ds(0, NB)], sidx.at[buf],
                                  isems[buf]).wait()
            pltpu.make_async_copy(eh_d.at[pl.ds(0, NB)], didx.at[buf],
                                  isems[buf]).wait()

        pltpu.sync_copy(z_hbm.at[pl.ds(s * RS, RS)],
                        acc_sh.at[pl.ds(s * RS, RS)])
        load_idx(0, 0)
        wait_idx(0)
        plsc.subcore_barrier()
        for k in range(NB):
            pltpu.async_copy(u_hbm.at[sidx.at[0, k]], rows_v.at[k], sems[k])
        load_idx(1, 1)

        def process_block(b, ib, nb):
            for k in range(NB):
                pltpu.make_async_copy(u_hbm.at[sidx.at[ib, k]],
                                      rows_v.at[k], sems[k]).wait()
                pltpu.sync_copy(rows_v.at[k],
                                acc_sh.at[didx.at[ib, k]], add=True)
                if k == 0:
                    @pl.when(nb < NBLK)
                    def _():
                        wait_idx(1 - ib)

                @pl.when(nb < NBLK)
                def _():
                    pltpu.async_copy(u_hbm.at[sidx.at[1 - ib, k]],
                                     rows_v.at[k], sems[k])

            @pl.when(nb + 1 < NBLK)
            def _():
                load_idx(nb + 1, ib)

        def body(j, carry):
            process_block(2 * j, 0, 2 * j + 1)
            process_block(2 * j + 1, 1, 2 * j + 2)
            return carry

        lax.fori_loop(0, NBLK // 2, body, 0)
        plsc.subcore_barrier()
        pltpu.sync_copy(acc_sh.at[pl.ds(s * RS, RS)],
                        out_hbm.at[c].at[pl.ds(s * RS, RS)])

    return _deg_sc, _prop_sc


def _prep_body(p16_ref, x_ref, w_ref, u_ref, dis_ref):
    p = p16_ref[...]
    cnt = p[0, :, :1] + p[1, :, :1]
    dis = lax.rsqrt(cnt + 1.0)
    u_ref[...] = jnp.dot(x_ref[...] * dis, w_ref[...],
                         preferred_element_type=F32)
    dis_ref[...] = dis


def _prep_tc(p16, xp, w0):
    return pl.pallas_call(
        _prep_body,
        grid=(NP // BR,),
        in_specs=[
            pl.BlockSpec((NC, BR, D), lambda i: (0, i, 0)),
            pl.BlockSpec((BR, D), lambda i: (i, 0)),
            pl.BlockSpec((D, D), lambda i: (0, 0)),
        ],
        out_specs=[
            pl.BlockSpec((BR, D), lambda i: (i, 0)),
            pl.BlockSpec((BR, 1), lambda i: (i, 0)),
        ],
        out_shape=[
            jax.ShapeDtypeStruct((NP, D), F32),
            jax.ShapeDtypeStruct((NP, 1), F32),
        ],
    )(p16, xp, w0)


def _layer_body(s_ref, u_ref, dis_ref, b_ref, w_ref, o_ref):
    sarr = s_ref[...]
    dis = dis_ref[...]
    u = u_ref[...]
    h = jnp.maximum(dis * (sarr[0] + sarr[1]) + dis * u + b_ref[...], 0.0)
    o_ref[...] = jnp.dot(dis * h, w_ref[...], preferred_element_type=F32)


def _layer_tc(s, u, dis, b, w_next):
    return pl.pallas_call(
        _layer_body,
        grid=(NP // BR,),
        in_specs=[
            pl.BlockSpec((NC, BR, D), lambda i: (0, i, 0)),
            pl.BlockSpec((BR, D), lambda i: (i, 0)),
            pl.BlockSpec((BR, 1), lambda i: (i, 0)),
            pl.BlockSpec((1, D), lambda i: (0, 0)),
            pl.BlockSpec((D, D), lambda i: (0, 0)),
        ],
        out_specs=pl.BlockSpec((BR, D), lambda i: (i, 0)),
        out_shape=jax.ShapeDtypeStruct((NP, D), F32),
    )(s, u, dis, b, w_next)


def _final_body(s_ref, u_ref, dis_ref, b_ref, bt_ref,
                f1w_ref, f1b_ref, f2w_ref, f2b_ref,
                o_ref, pooled, cnts):
    i = pl.program_id(0)

    @pl.when(i == 0)
    def _():
        pooled[...] = jnp.zeros_like(pooled)
        cnts[...] = jnp.zeros_like(cnts)

    sarr = s_ref[...]
    dis = dis_ref[...]
    u = u_ref[...]
    h = jnp.maximum(dis * (sarr[0] + sarr[1]) + dis * u + b_ref[...], 0.0)
    bt = bt_ref[...]
    m = (bt == lax.broadcasted_iota(jnp.int32, (BR, G), 1)).astype(F32)
    pooled[...] += lax.dot_general(m, h, (((0,), (0,)), ((), ())),
                                   preferred_element_type=F32)
    cnts[...] += lax.dot_general(m, jnp.ones_like(h), (((0,), (0,)), ((), ())),
                                 preferred_element_type=F32)

    @pl.when(i == NP // BR - 1)
    def _():
        pm = pooled[...] / jnp.maximum(cnts[...], 1.0)
        z = jnp.maximum(jnp.dot(pm, f1w_ref[...],
                                preferred_element_type=F32) + f1b_ref[...],
                        0.0)
        o_ref[...] = jnp.sum(z * f2w_ref[...], axis=1, keepdims=True) \
            + f2b_ref[...]


def _final_tc(s, u, dis, b, batchp, f1w, f1b, f2w, f2b):
    return pl.pallas_call(
        _final_body,
        grid=(NP // BR,),
        in_specs=[
            pl.BlockSpec((NC, BR, D), lambda i: (0, i, 0)),
            pl.BlockSpec((BR, D), lambda i: (i, 0)),
            pl.BlockSpec((BR, 1), lambda i: (i, 0)),
            pl.BlockSpec((1, D), lambda i: (0, 0)),
            pl.BlockSpec((BR, 1), lambda i: (i, 0)),
            pl.BlockSpec((D, D), lambda i: (0, 0)),
            pl.BlockSpec((1, D), lambda i: (0, 0)),
            pl.BlockSpec((1, D), lambda i: (0, 0)),
            pl.BlockSpec((1, 1), lambda i: (0, 0)),
        ],
        out_specs=pl.BlockSpec((G, 1), lambda i: (0, 0)),
        out_shape=jax.ShapeDtypeStruct((G, 1), F32),
        scratch_shapes=[
            pltpu.VMEM((G, D), F32),
            pltpu.VMEM((G, D), F32),
        ],
    )(s, u, dis, b, batchp, f1w, f1b, f2w, f2b)


def kernel(x, edge_index, batch, W0, b0, W1, b1, W2, b2,
           fc1_W, fc1_b, fc2_W, fc2_b):
    src = edge_index[0]
    dst = edge_index[1]
    padi = jnp.full((EP - E,), N, jnp.int32)
    srcf = jnp.concatenate([src, padi])
    dstf = jnp.concatenate([dst, padi])
    dstp = dstf.reshape(NC * NS, NCH, CH)
    srcq = srcf.reshape(NC * NS, NCHP, CHP)
    dstq = dstf.reshape(NC * NS, NCHP, CHP)

    xp = jnp.pad(x, ((0, NP - N), (0, 0)))
    zeros128 = jnp.zeros((NP, D), F32)
    ones128 = jnp.ones((CH, D), F32)
    batchp = jnp.pad(batch, (0, NP - N), constant_values=G).reshape(NP, 1)

    b0r = b0.reshape(1, D)
    b1r = b1.reshape(1, D)
    b2r = b2.reshape(1, D)
    f1b = fc1_b.reshape(1, D)
    f2w = fc2_W.reshape(1, D)
    f2b = fc2_b.reshape(1, 1)

    _deg_sc, _prop_sc = _sc_kernels()
    p16 = _deg_sc(dstp, ones128, zeros128)
    u0, dis = _prep_tc(p16, xp, W0)
    s0 = _prop_sc(u0, srcq, dstq, zeros128)
    u1 = _layer_tc(s0, u0, dis, b0r, W1)
    s1 = _prop_sc(u1, srcq, dstq, zeros128)
    u2 = _layer_tc(s1, u1, dis, b1r, W2)
    s2 = _prop_sc(u2, srcq, dstq, zeros128)
    return _final_tc(s2, u2, dis, b2r, batchp, fc1_W, f1b, f2w, f2b)


# speedup vs baseline: 8.2904x; 1.0730x over previous
import functools

import jax
import jax.numpy as jnp
from jax import lax
from jax.experimental import pallas as pl
from jax.experimental.pallas import tpu as pltpu
from jax.experimental.pallas import tpu_sc as plsc

N = 10000
NP = 10240
D = 128
E = 320000
EP = 327680
G = 16
NC = 2
NS = 16
CH = 128
NCH = EP // (NC * NS * CH)
CHP = 64
NCHP = EP // (NC * NS * CHP)
NB = 4
NBLK = NCHP // NB
B0 = 68
B1 = 12
RS = NP // NS
BR = 512
F32 = jnp.float32

@functools.lru_cache(maxsize=1)
def _sc_kernels():
    mesh = plsc.VectorSubcoreMesh(core_axis_name="c", subcore_axis_name="s")

    @functools.partial(
        pl.kernel,
        mesh=mesh,
        out_type=jax.ShapeDtypeStruct((NC, NP, D), F32),
        scratch_types=[
            pltpu.VMEM((NCH, CH), jnp.int32),
            pltpu.VMEM((CH, D), F32),
            pltpu.VMEM_SHARED((NP, D), F32),
        ],
    )
    def _deg_sc(dst_hbm, ones_hbm, z_hbm, out_hbm, idx_v, ones_v, acc_sh):
        c = lax.axis_index("c")
        s = lax.axis_index("s")
        wid = c * NS + s
        pltpu.sync_copy(dst_hbm.at[wid], idx_v)
        pltpu.sync_copy(ones_hbm, ones_v)
        pltpu.sync_copy(z_hbm.at[pl.ds(s * RS, RS)],
                        acc_sh.at[pl.ds(s * RS, RS)])
        plsc.subcore_barrier()

        def body(i, carry):
            pltpu.sync_copy(ones_v, acc_sh.at[idx_v.at[i]], add=True)
            return carry

        lax.fori_loop(0, NCH, body, 0)
        plsc.subcore_barrier()
        pltpu.sync_copy(acc_sh.at[pl.ds(s * RS, RS)],
                        out_hbm.at[c].at[pl.ds(s * RS, RS)])

    @functools.partial(
        pl.kernel,
        mesh=mesh,
        out_type=jax.ShapeDtypeStruct((NC, NP, D), F32),
        scratch_types=[
            pltpu.VMEM((2, NB, CHP), jnp.int32),
            pltpu.VMEM((2, NB, CHP), jnp.int32),
            pltpu.VMEM((NB, CHP, D), F32),
            pltpu.VMEM_SHARED((NP, D), F32),
        ] + [pltpu.SemaphoreType.DMA] * (NB + 2),
    )
    def _prop_sc(u_hbm, src_hbm, dst_hbm, z_hbm, out_hbm,
                 sidx, didx, rows_v, acc_sh, *sems):
        isems = sems[NB:]
        c = lax.axis_index("c")
        s = lax.axis_index("s")
        base_blk = jnp.where(c == 0, s * B0, 16 * B0 + s * B1)
        nblk = jnp.where(c == 0, B0, B1)

        def load_idx(g, buf):
            pltpu.async_copy(src_hbm.at[pl.ds(g * NB, NB)], sidx.at[buf],
                             isems[buf])
            pltpu.async_copy(dst_hbm.at[pl.ds(g * NB, NB)], didx.at[buf],
                             isems[buf])

        def wait_idx(buf):
            pltpu.make_async_copy(src_hbm.at[pl.ds(0, NB)], sidx.at[buf],
                                  isems[buf]).wait()
            pltpu.make_async_copy(dst_hbm.at[pl.ds(0, NB)], didx.at[buf],
                                  isems[buf]).wait()

        pltpu.sync_copy(z_hbm.at[pl.ds(s * RS, RS)],
                        acc_sh.at[pl.ds(s * RS, RS)])
        load_idx(base_blk, 0)
        wait_idx(0)
        plsc.subcore_barrier()
        for k in range(NB):
            pltpu.async_copy(u_hbm.at[sidx.at[0, k]], rows_v.at[k], sems[k])
        load_idx(base_blk + 1, 1)

        def process_block(jj, ib):
            for k in range(NB):
                pltpu.make_async_copy(u_hbm.at[sidx.at[ib, k]],
                                      rows_v.at[k], sems[k]).wait()
                pltpu.sync_copy(rows_v.at[k],
                                acc_sh.at[didx.at[ib, k]], add=True)
                if k == 0:
                    @pl.when(jj + 1 < nblk)
                    def _():
                        wait_idx(1 - ib)

                @pl.when(jj + 1 < nblk)
                def _():
                    pltpu.async_copy(u_hbm.at[sidx.at[1 - ib, k]],
                                     rows_v.at[k], sems[k])

            @pl.when(jj + 2 < nblk)
            def _():
                load_idx(base_blk + jj + 2, ib)

        def body(j, carry):
            process_block(2 * j, 0)
            process_block(2 * j + 1, 1)
            return carry

        lax.fori_loop(0, nblk // 2, body, 0)
        plsc.subcore_barrier()
        pltpu.sync_copy(acc_sh.at[pl.ds(s * RS, RS)],
                        out_hbm.at[c].at[pl.ds(s * RS, RS)])

    return _deg_sc, _prop_sc


def _prep_body(p16_ref, x_ref, w_ref, u_ref, dis_ref):
    p = p16_ref[...]
    cnt = p[0, :, :1] + p[1, :, :1]
    dis = lax.rsqrt(cnt + 1.0)
    u_ref[...] = jnp.dot(x_ref[...] * dis, w_ref[...],
                         preferred_element_type=F32)
    dis_ref[...] = dis


def _prep_tc(p16, xp, w0):
    return pl.pallas_call(
        _prep_body,
        grid=(NP // BR,),
        in_specs=[
            pl.BlockSpec((NC, BR, D), lambda i: (0, i, 0)),
            pl.BlockSpec((BR, D), lambda i: (i, 0)),
            pl.BlockSpec((D, D), lambda i: (0, 0)),
        ],
        out_specs=[
            pl.BlockSpec((BR, D), lambda i: (i, 0)),
            pl.BlockSpec((BR, 1), lambda i: (i, 0)),
        ],
        out_shape=[
            jax.ShapeDtypeStruct((NP, D), F32),
            jax.ShapeDtypeStruct((NP, 1), F32),
        ],
    )(p16, xp, w0)


def _layer_body(s_ref, u_ref, dis_ref, b_ref, w_ref, o_ref):
    sarr = s_ref[...]
    dis = dis_ref[...]
    u = u_ref[...]
    h = jnp.maximum(dis * (sarr[0] + sarr[1]) + dis * u + b_ref[...], 0.0)
    o_ref[...] = jnp.dot(dis * h, w_ref[...], preferred_element_type=F32)


def _layer_tc(s, u, dis, b, w_next):
    return pl.pallas_call(
        _layer_body,
        grid=(NP // BR,),
        in_specs=[
            pl.BlockSpec((NC, BR, D), lambda i: (0, i, 0)),
            pl.BlockSpec((BR, D), lambda i: (i, 0)),
            pl.BlockSpec((BR, 1), lambda i: (i, 0)),
            pl.BlockSpec((1, D), lambda i: (0, 0)),
            pl.BlockSpec((D, D), lambda i: (0, 0)),
        ],
        out_specs=pl.BlockSpec((BR, D), lambda i: (i, 0)),
        out_shape=jax.ShapeDtypeStruct((NP, D), F32),
    )(s, u, dis, b, w_next)


def _final_body(s_ref, u_ref, dis_ref, b_ref, bt_ref,
                f1w_ref, f1b_ref, f2w_ref, f2b_ref,
                o_ref, pooled, cnts):
    i = pl.program_id(0)

    @pl.when(i == 0)
    def _():
        pooled[...] = jnp.zeros_like(pooled)
        cnts[...] = jnp.zeros_like(cnts)

    sarr = s_ref[...]
    dis = dis_ref[...]
    u = u_ref[...]
    h = jnp.maximum(dis * (sarr[0] + sarr[1]) + dis * u + b_ref[...], 0.0)
    bt = bt_ref[...]
    m = (bt == lax.broadcasted_iota(jnp.int32, (BR, G), 1)).astype(F32)
    pooled[...] += lax.dot_general(m, h, (((0,), (0,)), ((), ())),
                                   preferred_element_type=F32)
    cnts[...] += lax.dot_general(m, jnp.ones_like(h), (((0,), (0,)), ((), ())),
                                 preferred_element_type=F32)

    @pl.when(i == NP // BR - 1)
    def _():
        pm = pooled[...] / jnp.maximum(cnts[...], 1.0)
        z = jnp.maximum(jnp.dot(pm, f1w_ref[...],
                                preferred_element_type=F32) + f1b_ref[...],
                        0.0)
        o_ref[...] = jnp.sum(z * f2w_ref[...], axis=1, keepdims=True) \
            + f2b_ref[...]


def _final_tc(s, u, dis, b, batchp, f1w, f1b, f2w, f2b):
    return pl.pallas_call(
        _final_body,
        grid=(NP // BR,),
        in_specs=[
            pl.BlockSpec((NC, BR, D), lambda i: (0, i, 0)),
            pl.BlockSpec((BR, D), lambda i: (i, 0)),
            pl.BlockSpec((BR, 1), lambda i: (i, 0)),
            pl.BlockSpec((1, D), lambda i: (0, 0)),
            pl.BlockSpec((BR, 1), lambda i: (i, 0)),
            pl.BlockSpec((D, D), lambda i: (0, 0)),
            pl.BlockSpec((1, D), lambda i: (0, 0)),
            pl.BlockSpec((1, D), lambda i: (0, 0)),
            pl.BlockSpec((1, 1), lambda i: (0, 0)),
        ],
        out_specs=pl.BlockSpec((G, 1), lambda i: (0, 0)),
        out_shape=jax.ShapeDtypeStruct((G, 1), F32),
        scratch_shapes=[
            pltpu.VMEM((G, D), F32),
            pltpu.VMEM((G, D), F32),
        ],
    )(s, u, dis, b, batchp, f1w, f1b, f2w, f2b)


def kernel(x, edge_index, batch, W0, b0, W1, b1, W2, b2,
           fc1_W, fc1_b, fc2_W, fc2_b):
    src = edge_index[0]
    dst = edge_index[1]
    padi = jnp.full((EP - E,), N, jnp.int32)
    srcf = jnp.concatenate([src, padi])
    dstf = jnp.concatenate([dst, padi])
    dstp = dstf.reshape(NC * NS, NCH, CH)
    srcq = srcf.reshape(EP // CHP, CHP)
    dstq = dstf.reshape(EP // CHP, CHP)

    xp = jnp.pad(x, ((0, NP - N), (0, 0)))
    zeros128 = jnp.zeros((NP, D), F32)
    ones128 = jnp.ones((CH, D), F32)
    batchp = jnp.pad(batch, (0, NP - N), constant_values=G).reshape(NP, 1)

    b0r = b0.reshape(1, D)
    b1r = b1.reshape(1, D)
    b2r = b2.reshape(1, D)
    f1b = fc1_b.reshape(1, D)
    f2w = fc2_W.reshape(1, D)
    f2b = fc2_b.reshape(1, 1)

    _deg_sc, _prop_sc = _sc_kernels()
    p16 = _deg_sc(dstp, ones128, zeros128)
    u0, dis = _prep_tc(p16, xp, W0)
    s0 = _prop_sc(u0, srcq, dstq, zeros128)
    u1 = _layer_tc(s0, u0, dis, b0r, W1)
    s1 = _prop_sc(u1, srcq, dstq, zeros128)
    u2 = _layer_tc(s1, u1, dis, b1r, W2)
    s2 = _prop_sc(u2, srcq, dstq, zeros128)
    return _final_tc(s2, u2, dis, b2r, batchp, fc1_W, f1b, f2w, f2b)
